# baseline (speedup 1.0000x reference)
import jax
import jax.numpy as jnp
from jax.experimental import pallas as pl
from jax.experimental.pallas import tpu as pltpu

TOKENS = 8192
INPUT_DIM = 4096
HIDDEN_DIM = 256
N_EXPERTS = 64

CHUNK = 128
NBUF = 8
NCHUNKS = TOKENS // CHUNK


def _gate_kernel(x_hbm, w1_ref, b1_ref, w2_ref, b2_ref,
                 gates_ref, idx_ref, buf, sem):
    def start_copy(c, b):
        pltpu.make_async_copy(
            x_hbm.at[pl.ds(c * CHUNK, CHUNK), :], buf.at[b], sem.at[b]
        ).start()

    for b in range(NBUF):
        start_copy(b, b)

    w1 = w1_ref[...]
    b1 = b1_ref[...]
    w2 = w2_ref[...]
    b2 = b2_ref[...]

    def body(c, _):
        b = jax.lax.rem(c, NBUF)
        pltpu.make_async_copy(
            x_hbm.at[pl.ds(c * CHUNK, CHUNK), :], buf.at[b], sem.at[b]
        ).wait()
        x = buf[b]
        h = jax.lax.dot_general(
            x, w1, (((1,), (1,)), ((), ())),
            preferred_element_type=jnp.float32)
        h = jnp.maximum(h + b1, 0.0)
        logits = jax.lax.dot_general(
            h, w2, (((1,), (1,)), ((), ())),
            preferred_element_type=jnp.float32)
        logits = logits + b2

        lanes = jax.lax.broadcasted_iota(jnp.int32, logits.shape, 1)
        l1 = jnp.max(logits, axis=-1, keepdims=True)
        i1 = jnp.argmax(logits, axis=-1).astype(jnp.int32)
        masked = jnp.where(lanes == i1[:, None], -jnp.inf, logits)
        l2 = jnp.max(masked, axis=-1, keepdims=True)
        i2 = jnp.argmax(masked, axis=-1).astype(jnp.int32)

        e = jnp.exp(l2 - l1)
        denom = 1.0 + e
        g1 = 1.0 / denom
        g2 = e / denom

        gates = jnp.where(lanes == i1[:, None], g1, 0.0)
        gates = jnp.where(lanes == i2[:, None], g2, gates)
        gates_ref[pl.ds(c * CHUNK, CHUNK), :] = gates
        idx_ref[pl.ds(c * CHUNK, CHUNK), :] = jnp.stack([i1, i2], axis=-1)

        nxt = c + NBUF

        @pl.when(nxt < NCHUNKS)
        def _():
            start_copy(nxt, b)

        return 0

    jax.lax.fori_loop(0, NCHUNKS, body, 0)


@jax.jit
def kernel(x, W1, b1, W2, b2):
    gates, idx = pl.pallas_call(
        _gate_kernel,
        in_specs=[
            pl.BlockSpec(memory_space=pl.ANY),
            pl.BlockSpec((HIDDEN_DIM, INPUT_DIM), lambda: (0, 0)),
            pl.BlockSpec((1, HIDDEN_DIM), lambda: (0, 0)),
            pl.BlockSpec((N_EXPERTS, HIDDEN_DIM), lambda: (0, 0)),
            pl.BlockSpec((1, N_EXPERTS), lambda: (0, 0)),
        ],
        out_specs=[
            pl.BlockSpec((TOKENS, N_EXPERTS), lambda: (0, 0)),
            pl.BlockSpec((TOKENS, 2), lambda: (0, 0)),
        ],
        out_shape=[
            jax.ShapeDtypeStruct((TOKENS, N_EXPERTS), jnp.float32),
            jax.ShapeDtypeStruct((TOKENS, 2), jnp.int32),
        ],
        scratch_shapes=[
            pltpu.VMEM((NBUF, CHUNK, INPUT_DIM), jnp.float32),
            pltpu.SemaphoreType.DMA((NBUF,)),
        ],
    )(x, W1, b1.reshape(1, HIDDEN_DIM), W2, b2.reshape(1, N_EXPERTS))
    return gates, idx


# speedup vs baseline: 1.5632x; 1.5632x over previous
import jax
import jax.numpy as jnp
from jax.experimental import pallas as pl
from jax.experimental.pallas import tpu as pltpu

TOKENS = 8192
INPUT_DIM = 4096
HIDDEN_DIM = 256
N_EXPERTS = 64

GROUP = 512
SUB = 4
NGROUP = 4
SUBROWS = GROUP // SUB
NGROUPS = TOKENS // GROUP


def _gate_kernel(x_hbm, w1_ref, b1_ref, w2_ref, b2_ref,
                 gates_ref, idx_ref, buf, sem):
    def start_group(g, b):
        for s in range(SUB):
            pltpu.make_async_copy(
                x_hbm.at[pl.ds(g * GROUP + s * SUBROWS, SUBROWS), :],
                buf.at[b, pl.ds(s * SUBROWS, SUBROWS), :],
                sem.at[b, s],
            ).start()

    def wait_group(g, b):
        for s in range(SUB):
            pltpu.make_async_copy(
                x_hbm.at[pl.ds(g * GROUP + s * SUBROWS, SUBROWS), :],
                buf.at[b, pl.ds(s * SUBROWS, SUBROWS), :],
                sem.at[b, s],
            ).wait()

    for b in range(NGROUP):
        start_group(b, b)

    w1 = w1_ref[...]
    b1 = b1_ref[...]
    w2 = w2_ref[...]
    b2 = b2_ref[...]

    def body(c, _):
        b = jax.lax.rem(c, NGROUP)
        wait_group(c, b)
        x = buf[b]
        h = jax.lax.dot_general(
            x, w1, (((1,), (1,)), ((), ())),
            preferred_element_type=jnp.float32)
        h = jnp.maximum(h + b1, 0.0)
        logits = jax.lax.dot_general(
            h, w2, (((1,), (1,)), ((), ())),
            preferred_element_type=jnp.float32)
        logits = logits + b2

        lanes = jax.lax.broadcasted_iota(jnp.int32, logits.shape, 1)
        l1 = jnp.max(logits, axis=-1, keepdims=True)
        i1 = jnp.argmax(logits, axis=-1).astype(jnp.int32)
        masked = jnp.where(lanes == i1[:, None], -jnp.inf, logits)
        l2 = jnp.max(masked, axis=-1, keepdims=True)
        i2 = jnp.argmax(masked, axis=-1).astype(jnp.int32)

        e = jnp.exp(l2 - l1)
        denom = 1.0 + e
        g1 = 1.0 / denom
        g2 = e / denom

        gates = jnp.where(lanes == i1[:, None], g1, 0.0)
        gates = jnp.where(lanes == i2[:, None], g2, gates)
        gates_ref[pl.ds(c * GROUP, GROUP), :] = gates
        idx_ref[pl.ds(c * GROUP, GROUP), :] = jnp.stack([i1, i2], axis=-1)

        nxt = c + NGROUP

        @pl.when(nxt < NGROUPS)
        def _():
            start_group(nxt, b)

        return 0

    jax.lax.fori_loop(0, NGROUPS, body, 0)


@jax.jit
def kernel(x, W1, b1, W2, b2):
    gates, idx = pl.pallas_call(
        _gate_kernel,
        in_specs=[
            pl.BlockSpec(memory_space=pl.ANY),
            pl.BlockSpec((HIDDEN_DIM, INPUT_DIM), lambda: (0, 0)),
            pl.BlockSpec((1, HIDDEN_DIM), lambda: (0, 0)),
            pl.BlockSpec((N_EXPERTS, HIDDEN_DIM), lambda: (0, 0)),
            pl.BlockSpec((1, N_EXPERTS), lambda: (0, 0)),
        ],
        out_specs=[
            pl.BlockSpec((TOKENS, N_EXPERTS), lambda: (0, 0)),
            pl.BlockSpec((TOKENS, 2), lambda: (0, 0)),
        ],
        out_shape=[
            jax.ShapeDtypeStruct((TOKENS, N_EXPERTS), jnp.float32),
            jax.ShapeDtypeStruct((TOKENS, 2), jnp.int32),
        ],
        scratch_shapes=[
            pltpu.VMEM((NGROUP, GROUP, INPUT_DIM), jnp.float32),
            pltpu.SemaphoreType.DMA((NGROUP, SUB)),
        ],
    )(x, W1, b1.reshape(1, HIDDEN_DIM), W2, b2.reshape(1, N_EXPERTS))
    return gates, idx


# speedup vs baseline: 1.5686x; 1.0034x over previous
import jax
import jax.numpy as jnp
from jax.experimental import pallas as pl
from jax.experimental.pallas import tpu as pltpu

TOKENS = 8192
INPUT_DIM = 4096
HIDDEN_DIM = 256
N_EXPERTS = 64

GROUP = 512
SUB = 4
NGROUP = 4
SUBROWS = GROUP // SUB
NGROUPS = TOKENS // GROUP


def _gate_kernel(x_hbm, w1_ref, b1_ref, w2_ref, b2_ref,
                 gates_ref, idx_ref, buf, sem):
    def start_group(g, b):
        for s in range(SUB):
            pltpu.make_async_copy(
                x_hbm.at[pl.ds(g * GROUP + s * SUBROWS, SUBROWS), :],
                buf.at[b, s],
                sem.at[b, s],
            ).start()

    def wait_group(g, b):
        for s in range(SUB):
            pltpu.make_async_copy(
                x_hbm.at[pl.ds(g * GROUP + s * SUBROWS, SUBROWS), :],
                buf.at[b, s],
                sem.at[b, s],
            ).wait()

    for b in range(NGROUP):
        start_group(b, b)

    w1 = w1_ref[...]
    b1 = b1_ref[...]
    w2 = w2_ref[...]
    b2 = b2_ref[...]

    def body(c, _):
        b = jax.lax.rem(c, NGROUP)
        wait_group(c, b)
        x = buf[b].reshape(GROUP, INPUT_DIM)
        h = jax.lax.dot_general(
            x, w1, (((1,), (1,)), ((), ())),
            preferred_element_type=jnp.float32)
        h = jnp.maximum(h + b1, 0.0)
        logits = jax.lax.dot_general(
            h, w2, (((1,), (1,)), ((), ())),
            preferred_element_type=jnp.float32)
        logits = logits + b2

        lanes = jax.lax.broadcasted_iota(jnp.int32, logits.shape, 1)
        l1 = jnp.max(logits, axis=-1, keepdims=True)
        i1 = jnp.argmax(logits, axis=-1).astype(jnp.int32)
        masked = jnp.where(lanes == i1[:, None], -jnp.inf, logits)
        l2 = jnp.max(masked, axis=-1, keepdims=True)
        i2 = jnp.argmax(masked, axis=-1).astype(jnp.int32)

        e = jnp.exp(l2 - l1)
        denom = 1.0 + e
        g1 = 1.0 / denom
        g2 = e / denom

        gates = jnp.where(lanes == i1[:, None], g1, 0.0)
        gates = jnp.where(lanes == i2[:, None], g2, gates)
        gates_ref[pl.ds(c * GROUP, GROUP), :] = gates
        idx_ref[pl.ds(c * GROUP, GROUP), :] = jnp.stack([i1, i2], axis=-1)

        nxt = c + NGROUP

        @pl.when(nxt < NGROUPS)
        def _():
            start_group(nxt, b)

        return 0

    jax.lax.fori_loop(0, NGROUPS, body, 0)


@jax.jit
def kernel(x, W1, b1, W2, b2):
    gates, idx = pl.pallas_call(
        _gate_kernel,
        in_specs=[
            pl.BlockSpec(memory_space=pl.ANY),
            pl.BlockSpec((HIDDEN_DIM, INPUT_DIM), lambda: (0, 0)),
            pl.BlockSpec((1, HIDDEN_DIM), lambda: (0, 0)),
            pl.BlockSpec((N_EXPERTS, HIDDEN_DIM), lambda: (0, 0)),
            pl.BlockSpec((1, N_EXPERTS), lambda: (0, 0)),
        ],
        out_specs=[
            pl.BlockSpec((TOKENS, N_EXPERTS), lambda: (0, 0)),
            pl.BlockSpec((TOKENS, 2), lambda: (0, 0)),
        ],
        out_shape=[
            jax.ShapeDtypeStruct((TOKENS, N_EXPERTS), jnp.float32),
            jax.ShapeDtypeStruct((TOKENS, 2), jnp.int32),
        ],
        scratch_shapes=[
            pltpu.VMEM((NGROUP, SUB, SUBROWS, INPUT_DIM), jnp.float32),
            pltpu.SemaphoreType.DMA((NGROUP, SUB)),
        ],
    )(x, W1, b1.reshape(1, HIDDEN_DIM), W2, b2.reshape(1, N_EXPERTS))
    return gates, idx
